# baseline (speedup 1.0000x reference)
import functools

import jax
import jax.numpy as jnp
from jax import lax
from jax.experimental import pallas as pl
from jax.experimental.pallas import tpu as pltpu
from jax.experimental.pallas import tpu_sc as plsc

R, C, V = 16384, 128, 100000
TOTAL = R * C
NC, NS, L = 2, 16, 16
NW = NC * NS
PER_W = TOTAL // NW
CHUNK = 4096
NCHUNK = PER_W // CHUNK
GRP = C // L

_MESH = plsc.VectorSubcoreMesh(
    core_axis_name="c", subcore_axis_name="s", num_cores=NC, num_subcores=NS
)


def _body(x_hbm, ids_hbm, out_hbm, ids_v, fidx_v, gat_v, sem_i, sem_g):
    wid = lax.axis_index("s") * NC + lax.axis_index("c")
    iota = lax.iota(jnp.int32, 16)
    jvecs = [iota + (u * L) for u in range(GRP)]
    base0 = wid * PER_W

    def chunk_base(ci):
        return base0 + ci * CHUNK

    def load_ids(ci, b):
        pltpu.async_copy(
            ids_hbm.at[pl.ds(chunk_base(ci), CHUNK)], ids_v[b], sem_i[b]
        )

    def fix_and_fire(b):
        def body(r, carry):
            for u in range(GRP):
                sl = pl.ds(r * C + u * L, L)
                fidx_v[b][sl] = ids_v[b][sl] * C + jvecs[u]
            return carry

        lax.fori_loop(0, CHUNK // C, body, 0)
        pltpu.async_copy(x_hbm.at[fidx_v[b]], gat_v[b], sem_g[b])

    def drain_and_store(ci, b):
        pltpu.make_async_copy(
            out_hbm.at[pl.ds(chunk_base(ci), CHUNK)], gat_v[b], sem_g[b]
        ).wait()
        pltpu.sync_copy(gat_v[b], out_hbm.at[pl.ds(chunk_base(ci), CHUNK)])

    load_ids(0, 0)
    for ci in range(NCHUNK):
        b = ci % 2
        pltpu.make_async_copy(
            ids_hbm.at[pl.ds(chunk_base(ci), CHUNK)], ids_v[b], sem_i[b]
        ).wait()
        if ci + 1 < NCHUNK:
            load_ids(ci + 1, 1 - b)
        fix_and_fire(b)
        if ci > 0:
            drain_and_store(ci - 1, 1 - b)
    drain_and_store(NCHUNK - 1, (NCHUNK - 1) % 2)


@functools.partial(
    pl.kernel,
    out_type=jax.ShapeDtypeStruct((TOTAL,), jnp.float32),
    mesh=_MESH,
    scratch_types=[
        [pltpu.VMEM((CHUNK,), jnp.int32) for _ in range(2)],
        [pltpu.VMEM((CHUNK,), jnp.int32) for _ in range(2)],
        [pltpu.VMEM((CHUNK,), jnp.float32) for _ in range(2)],
        [pltpu.SemaphoreType.DMA for _ in range(2)],
        [pltpu.SemaphoreType.DMA for _ in range(2)],
    ],
)
def _gather_sc(x_flat, ids_flat, out, ids_v, fidx_v, gat_v, sem_i, sem_g):
    _body(x_flat, ids_flat, out, ids_v, fidx_v, gat_v, sem_i, sem_g)


def kernel(x, ids):
    out = _gather_sc(x.reshape(-1), ids.astype(jnp.int32).reshape(-1))
    return out.reshape(R, C)


# speedup vs baseline: 1.0220x; 1.0220x over previous
import functools

import jax
import jax.numpy as jnp
from jax import lax
from jax.experimental import pallas as pl
from jax.experimental.pallas import tpu as pltpu
from jax.experimental.pallas import tpu_sc as plsc

R, C, V = 16384, 128, 100000
TOTAL = R * C
NC, NS, L = 2, 16, 16
NW = NC * NS
PER_W = TOTAL // NW
CHUNK = 8192
CHUNKS = (1024,) + (CHUNK,) * 7 + (7168,)
NCHUNK = len(CHUNKS)
GRP = C // L

_MESH = plsc.VectorSubcoreMesh(
    core_axis_name="c", subcore_axis_name="s", num_cores=NC, num_subcores=NS
)


def _body(x_hbm, ids_hbm, out_hbm, ids_v, fidx_v, gat_v, sem_i, sem_g):
    wid = lax.axis_index("s") * NC + lax.axis_index("c")
    iota = lax.iota(jnp.int32, 16)
    jvecs = [iota + (u * L) for u in range(GRP)]
    base0 = wid * PER_W

    def chunk_base(ci):
        return base0 + sum(CHUNKS[:ci])

    def load_ids(ci, b):
        n = CHUNKS[ci]
        pltpu.async_copy(
            ids_hbm.at[pl.ds(chunk_base(ci), n)], ids_v[b].at[pl.ds(0, n)],
            sem_i[b],
        )

    def wait_ids(ci, b):
        n = CHUNKS[ci]
        pltpu.make_async_copy(
            ids_hbm.at[pl.ds(chunk_base(ci), n)], ids_v[b].at[pl.ds(0, n)],
            sem_i[b],
        ).wait()

    def fix_and_fire(ci, b):
        n = CHUNKS[ci]

        def body(r, carry):
            for u in range(GRP):
                sl = pl.ds(r * C + u * L, L)
                fidx_v[b][sl] = ids_v[b][sl] * C + jvecs[u]
            return carry

        lax.fori_loop(0, n // C, body, 0)
        pltpu.async_copy(
            x_hbm.at[fidx_v[b].at[pl.ds(0, n)]], gat_v[b].at[pl.ds(0, n)],
            sem_g[b],
        )

    def drain_and_store(ci, b):
        n = CHUNKS[ci]
        pltpu.make_async_copy(
            out_hbm.at[pl.ds(chunk_base(ci), n)], gat_v[b].at[pl.ds(0, n)],
            sem_g[b],
        ).wait()
        pltpu.sync_copy(
            gat_v[b].at[pl.ds(0, n)], out_hbm.at[pl.ds(chunk_base(ci), n)]
        )

    load_ids(0, 0)
    for ci in range(NCHUNK):
        b = ci % 2
        wait_ids(ci, b)
        if ci + 1 < NCHUNK:
            load_ids(ci + 1, 1 - b)
        fix_and_fire(ci, b)
        if ci > 0:
            drain_and_store(ci - 1, 1 - b)
    drain_and_store(NCHUNK - 1, (NCHUNK - 1) % 2)


@functools.partial(
    pl.kernel,
    out_type=jax.ShapeDtypeStruct((TOTAL,), jnp.float32),
    mesh=_MESH,
    scratch_types=[
        [pltpu.VMEM((CHUNK,), jnp.int32) for _ in range(2)],
        [pltpu.VMEM((CHUNK,), jnp.int32) for _ in range(2)],
        [pltpu.VMEM((CHUNK,), jnp.float32) for _ in range(2)],
        [pltpu.SemaphoreType.DMA for _ in range(2)],
        [pltpu.SemaphoreType.DMA for _ in range(2)],
    ],
)
def _gather_sc(x_flat, ids_flat, out, ids_v, fidx_v, gat_v, sem_i, sem_g):
    _body(x_flat, ids_flat, out, ids_v, fidx_v, gat_v, sem_i, sem_g)


def kernel(x, ids):
    out = _gather_sc(x.reshape(-1), ids.astype(jnp.int32).reshape(-1))
    return out.reshape(R, C)
